# baseline (speedup 1.0000x reference)
import functools
import math

import jax
import jax.numpy as jnp
from jax.experimental import pallas as pl
from jax.experimental.pallas import tpu as pltpu

EPS = 1e-5
KS = 3
IMGS_PER_STEP_1 = 4
IMGS_PER_STEP_2 = 4


def _conv_stats_kernel(x_ref, w_ref, mask_ref, ones_ref, y_ref, stats_ref,
                       xx0_ref, xx1_ref, p0_ref, p1_ref, *, cin, hw, width):
    g = width + 1
    m = IMGS_PER_STEP_1

    def fill(img):
        xx_ref = xx0_ref if img % 2 == 0 else xx1_ref
        patch_ref = p0_ref if img % 2 == 0 else p1_ref
        xx_ref[:, pl.ds(0, g)] = jnp.zeros((cin, g), jnp.bfloat16)
        xx_ref[:, pl.ds(g + hw, g)] = jnp.zeros((cin, g), jnp.bfloat16)
        xx_ref[:, pl.ds(g, hw)] = x_ref[img].astype(jnp.bfloat16)
        for ky in range(KS):
            for kx in range(KS):
                tap = ky * KS + kx
                t = xx_ref[:, pl.ds(ky * width + kx, hw)]
                if kx == 0:
                    t = t * mask_ref[0:1, :]
                elif kx == KS - 1:
                    t = t * mask_ref[1:2, :]
                patch_ref[pl.ds(tap * cin, cin), :] = t

    fill(0)
    for img in range(m):
        patch_ref = p0_ref if img % 2 == 0 else p1_ref
        y = jnp.dot(w_ref[...], patch_ref[...],
                    preferred_element_type=jnp.float32)
        if img + 1 < m:
            fill(img + 1)
        yb = y.astype(jnp.bfloat16)
        y_ref[img] = yb
        y2b = (y * y).astype(jnp.bfloat16)
        s0 = jnp.dot(yb, ones_ref[...], preferred_element_type=jnp.float32)
        s1 = jnp.dot(y2b, ones_ref[...], preferred_element_type=jnp.float32)
        stats_ref[img, :, 0:1] = s0[:, 0:1]
        stats_ref[img, :, 1:2] = s1[:, 0:1]


def _bn_relu_kernel(y_ref, stats_ref, g_ref, b_ref, o_ref, *, cnt):
    s = jnp.sum(stats_ref[...], axis=0)
    mean = s[:, 0:1] / cnt
    var = jnp.maximum(s[:, 1:2] / cnt - mean * mean, 0.0)
    inv = jax.lax.rsqrt(var + EPS)
    g_col = g_ref[...].T
    b_col = b_ref[...].T
    scale = g_col * inv
    bias = b_col - mean * scale
    y = y_ref[...].astype(jnp.float32)
    o_ref[...] = jnp.maximum(y * scale + bias, 0.0)


def kernel(x, weight, gamma, beta):
    n, cin, h, width = x.shape
    cout = weight.shape[0]
    hw = h * width
    flat = hw + 2 * (width + 1)

    xf = x.reshape(n, cin, hw)

    w_mat = weight.transpose(0, 2, 3, 1).reshape(cout, KS * KS * cin)
    w_mat = w_mat.astype(jnp.bfloat16)

    col = jnp.arange(hw, dtype=jnp.int32) % width
    mask = jnp.stack([col != 0, col != width - 1]).astype(jnp.bfloat16)
    ones = jnp.ones((hw, 8), jnp.bfloat16)

    m1 = IMGS_PER_STEP_1
    kern = functools.partial(_conv_stats_kernel, cin=cin, hw=hw, width=width)
    y_flat, stats = pl.pallas_call(
        kern,
        grid=(n // m1,),
        in_specs=[pl.BlockSpec((m1, cin, hw), lambda i: (i, 0, 0)),
                  pl.BlockSpec((cout, KS * KS * cin), lambda i: (0, 0)),
                  pl.BlockSpec((2, hw), lambda i: (0, 0)),
                  pl.BlockSpec((hw, 8), lambda i: (0, 0))],
        out_specs=[pl.BlockSpec((m1, cout, hw), lambda i: (i, 0, 0)),
                   pl.BlockSpec((m1, cout, 2), lambda i: (i, 0, 0))],
        out_shape=[jax.ShapeDtypeStruct((n, cout, hw), jnp.bfloat16),
                   jax.ShapeDtypeStruct((n, cout, 2), jnp.float32)],
        scratch_shapes=[pltpu.VMEM((cin, flat), jnp.bfloat16),
                        pltpu.VMEM((cin, flat), jnp.bfloat16),
                        pltpu.VMEM((KS * KS * cin, hw), jnp.bfloat16),
                        pltpu.VMEM((KS * KS * cin, hw), jnp.bfloat16)],
        compiler_params=pltpu.CompilerParams(
            dimension_semantics=("parallel",)),
    )(xf, w_mat, mask, ones)

    m2 = IMGS_PER_STEP_2
    out_flat = pl.pallas_call(
        functools.partial(_bn_relu_kernel, cnt=float(n * hw)),
        grid=(n // m2,),
        in_specs=[pl.BlockSpec((m2, cout, hw), lambda i: (i, 0, 0)),
                  pl.BlockSpec((n, cout, 2), lambda i: (0, 0, 0)),
                  pl.BlockSpec((1, cout), lambda i: (0, 0)),
                  pl.BlockSpec((1, cout), lambda i: (0, 0))],
        out_specs=pl.BlockSpec((m2, cout, hw), lambda i: (i, 0, 0)),
        out_shape=jax.ShapeDtypeStruct((n, cout, hw), x.dtype),
        compiler_params=pltpu.CompilerParams(
            dimension_semantics=("parallel",)),
    )(y_flat, stats, gamma.astype(jnp.float32).reshape(1, cout),
      beta.astype(jnp.float32).reshape(1, cout))

    return out_flat.reshape(n, cout, h, width)


# speedup vs baseline: 1.0305x; 1.0305x over previous
import functools
import math

import jax
import jax.numpy as jnp
from jax.experimental import pallas as pl
from jax.experimental.pallas import tpu as pltpu

EPS = 1e-5
KS = 3
IMGS_PER_STEP_1 = 4
IMGS_PER_STEP_2 = 4


def _conv_stats_kernel(x_ref, w_ref, mask_ref, y_ref, stats_ref,
                       xx0_ref, xx1_ref, p0_ref, p1_ref, *, cin, hw, width):
    g = width + 1
    m = IMGS_PER_STEP_1

    def fill(img):
        xx_ref = xx0_ref if img % 2 == 0 else xx1_ref
        patch_ref = p0_ref if img % 2 == 0 else p1_ref
        xx_ref[:, pl.ds(0, g)] = jnp.zeros((cin, g), jnp.bfloat16)
        xx_ref[:, pl.ds(g + hw, g)] = jnp.zeros((cin, g), jnp.bfloat16)
        xx_ref[:, pl.ds(g, hw)] = x_ref[img].astype(jnp.bfloat16)
        for ky in range(KS):
            for kx in range(KS):
                tap = ky * KS + kx
                t = xx_ref[:, pl.ds(ky * width + kx, hw)]
                if kx == 0:
                    t = t * mask_ref[0:1, :]
                elif kx == KS - 1:
                    t = t * mask_ref[1:2, :]
                patch_ref[pl.ds(tap * cin, cin), :] = t

    fill(0)
    for img in range(m):
        patch_ref = p0_ref if img % 2 == 0 else p1_ref
        y = jnp.dot(w_ref[...], patch_ref[...],
                    preferred_element_type=jnp.float32)
        if img + 1 < m:
            fill(img + 1)
        y_ref[img] = y.astype(jnp.bfloat16)
        stats_ref[img, :, 0:1] = jnp.sum(y, axis=1, keepdims=True)
        stats_ref[img, :, 1:2] = jnp.sum(y * y, axis=1, keepdims=True)


def _bn_relu_kernel(y_ref, stats_ref, g_ref, b_ref, o_ref, *, cnt):
    s = jnp.sum(stats_ref[...], axis=0)
    mean = s[:, 0:1] / cnt
    var = jnp.maximum(s[:, 1:2] / cnt - mean * mean, 0.0)
    inv = jax.lax.rsqrt(var + EPS)
    g_col = g_ref[...].T
    b_col = b_ref[...].T
    scale = g_col * inv
    bias = b_col - mean * scale
    y = y_ref[...].astype(jnp.float32)
    o_ref[...] = jnp.maximum(y * scale + bias, 0.0)


def kernel(x, weight, gamma, beta):
    n, cin, h, width = x.shape
    cout = weight.shape[0]
    hw = h * width
    flat = hw + 2 * (width + 1)

    xf = x.reshape(n, cin, hw)

    w_mat = weight.transpose(0, 2, 3, 1).reshape(cout, KS * KS * cin)
    w_mat = w_mat.astype(jnp.bfloat16)

    col = jnp.arange(hw, dtype=jnp.int32) % width
    mask = jnp.stack([col != 0, col != width - 1]).astype(jnp.bfloat16)

    m1 = IMGS_PER_STEP_1
    kern = functools.partial(_conv_stats_kernel, cin=cin, hw=hw, width=width)
    y_flat, stats = pl.pallas_call(
        kern,
        grid=(n // m1,),
        in_specs=[pl.BlockSpec((m1, cin, hw), lambda i: (i, 0, 0)),
                  pl.BlockSpec((cout, KS * KS * cin), lambda i: (0, 0)),
                  pl.BlockSpec((2, hw), lambda i: (0, 0))],
        out_specs=[pl.BlockSpec((m1, cout, hw), lambda i: (i, 0, 0)),
                   pl.BlockSpec((m1, cout, 2), lambda i: (i, 0, 0))],
        out_shape=[jax.ShapeDtypeStruct((n, cout, hw), jnp.bfloat16),
                   jax.ShapeDtypeStruct((n, cout, 2), jnp.float32)],
        scratch_shapes=[pltpu.VMEM((cin, flat), jnp.bfloat16),
                        pltpu.VMEM((cin, flat), jnp.bfloat16),
                        pltpu.VMEM((KS * KS * cin, hw), jnp.bfloat16),
                        pltpu.VMEM((KS * KS * cin, hw), jnp.bfloat16)],
        compiler_params=pltpu.CompilerParams(
            dimension_semantics=("parallel",)),
    )(xf, w_mat, mask)

    m2 = IMGS_PER_STEP_2
    out_flat = pl.pallas_call(
        functools.partial(_bn_relu_kernel, cnt=float(n * hw)),
        grid=(n // m2,),
        in_specs=[pl.BlockSpec((m2, cout, hw), lambda i: (i, 0, 0)),
                  pl.BlockSpec((n, cout, 2), lambda i: (0, 0, 0)),
                  pl.BlockSpec((1, cout), lambda i: (0, 0)),
                  pl.BlockSpec((1, cout), lambda i: (0, 0))],
        out_specs=pl.BlockSpec((m2, cout, hw), lambda i: (i, 0, 0)),
        out_shape=jax.ShapeDtypeStruct((n, cout, hw), x.dtype),
        compiler_params=pltpu.CompilerParams(
            dimension_semantics=("parallel",)),
    )(y_flat, stats, gamma.astype(jnp.float32).reshape(1, cout),
      beta.astype(jnp.float32).reshape(1, cout))

    return out_flat.reshape(n, cout, h, width)
